# Initial kernel scaffold; baseline (speedup 1.0000x reference)
import jax
import jax.numpy as jnp
from jax.experimental import pallas as pl


def kernel(x, edge_index, edge_weight, W):
    raise NotImplementedError("write your pallas kernel here")


# speedup vs baseline: 3.0538x; 3.0538x over previous
import functools

import jax
import jax.numpy as jnp
from jax import lax
from jax.experimental import pallas as pl
from jax.experimental.pallas import tpu as pltpu
from jax.experimental.pallas import tpu_sc as plsc

N = 10000
E = 320000
D = 128

NC = 2
NS = 16
NW = NC * NS
EPW = E // NW
K = 80
CH = EPW // K
RPT = 632
NP = NS * RPT
MB = 1000


def _mm_body(x_ref, w_ref, o_ref):
    o_ref[...] = jnp.dot(x_ref[...], w_ref[...],
                         preferred_element_type=jnp.float32)


def _combine_body(a_ref, b_ref, o_ref):
    o_ref[...] = jnp.maximum(a_ref[0] + b_ref[0], 0.0)


_mesh = plsc.VectorSubcoreMesh(core_axis_name="c", subcore_axis_name="s")


@functools.partial(
    pl.kernel,
    mesh=_mesh,
    out_type=jax.ShapeDtypeStruct((NC, NP, D), jnp.float32),
    scratch_types=[
        pltpu.VMEM((K,), jnp.int32),
        pltpu.VMEM((K,), jnp.int32),
        pltpu.VMEM((K, 16), jnp.float32),
        pltpu.VMEM((K, D), jnp.float32),
        pltpu.VMEM_SHARED((NP, D), jnp.float32),
        pltpu.SemaphoreType.DMA,
    ],
)
def _sc_scatter(presup, src, dst, w, zeros, out,
                src_v, dst_v, w_v, rows_v, acc, sem):
    cid = lax.axis_index("c")
    sid = lax.axis_index("s")
    wid = cid * NS + sid

    pltpu.sync_copy(zeros, acc.at[pl.ds(sid * RPT, RPT)])
    plsc.subcore_barrier()

    def chunk_body(j, carry):
        base = wid * EPW + j * K
        pltpu.sync_copy(src.at[pl.ds(base, K)], src_v)
        pltpu.sync_copy(dst.at[pl.ds(base, K)], dst_v)
        pltpu.sync_copy(w.at[pl.ds(base, K)], w_v)
        pltpu.async_copy(presup.at[src_v], rows_v, sem).wait()

        def scale_body(k, c2):
            wv = w_v[k, :]
            for c in range(D // 16):
                sl = pl.ds(c * 16, 16)
                rows_v[k, sl] = rows_v[k, sl] * wv
            return c2
        lax.fori_loop(0, K, scale_body, 0)

        pltpu.sync_copy(rows_v, acc.at[dst_v], add=True)
        return carry
    lax.fori_loop(0, CH, chunk_body, 0)

    plsc.subcore_barrier()
    pltpu.sync_copy(acc.at[pl.ds(sid * RPT, RPT)],
                    out.at[cid, pl.ds(sid * RPT, RPT)])


def kernel(x, edge_index, edge_weight, W):
    pre_sup = pl.pallas_call(
        _mm_body,
        grid=(N // MB,),
        in_specs=[
            pl.BlockSpec((MB, D), lambda i: (i, 0)),
            pl.BlockSpec((D, D), lambda i: (0, 0)),
        ],
        out_specs=pl.BlockSpec((MB, D), lambda i: (i, 0)),
        out_shape=jax.ShapeDtypeStruct((N, D), jnp.float32),
    )(x, W)

    src = edge_index[0].astype(jnp.int32)
    dst = edge_index[1].astype(jnp.int32)
    zeros = jnp.zeros((RPT, D), jnp.float32)
    w_splat = jnp.broadcast_to(edge_weight.astype(jnp.float32)[:, None],
                               (E, 16))

    partials = _sc_scatter(pre_sup, src, dst, w_splat, zeros)

    out = pl.pallas_call(
        _combine_body,
        grid=(N // MB,),
        in_specs=[
            pl.BlockSpec((1, MB, D), lambda i: (0, i, 0)),
            pl.BlockSpec((1, MB, D), lambda i: (1, i, 0)),
        ],
        out_specs=pl.BlockSpec((MB, D), lambda i: (i, 0)),
        out_shape=jax.ShapeDtypeStruct((N, D), jnp.float32),
    )(partials, partials)
    return out


# speedup vs baseline: 5.6949x; 1.8649x over previous
import functools

import jax
import jax.numpy as jnp
from jax import lax
from jax.experimental import pallas as pl
from jax.experimental.pallas import tpu as pltpu
from jax.experimental.pallas import tpu_sc as plsc

N = 10000
E = 320000
D = 128

NC = 2
NS = 16
NW = NC * NS
EPW = E // NW
K = 80
CH = EPW // K
RPT = 632
NP = NS * RPT
MB = 1000


def _mm_body(x_ref, w_ref, o_ref):
    o_ref[...] = jnp.dot(x_ref[...], w_ref[...],
                         preferred_element_type=jnp.float32)


def _combine_body(a_ref, b_ref, o_ref):
    o_ref[...] = jnp.maximum(a_ref[0] + b_ref[0], 0.0)


_mesh = plsc.VectorSubcoreMesh(core_axis_name="c", subcore_axis_name="s")


@functools.partial(
    pl.kernel,
    mesh=_mesh,
    out_type=jax.ShapeDtypeStruct((NC, NP, D), jnp.float32),
    scratch_types=[
        pltpu.VMEM((2, K), jnp.int32),
        pltpu.VMEM((2, K), jnp.int32),
        pltpu.VMEM((2, K, 16), jnp.float32),
        pltpu.VMEM((2, K, D), jnp.float32),
        pltpu.VMEM_SHARED((NP, D), jnp.float32),
        pltpu.SemaphoreType.DMA,
        pltpu.SemaphoreType.DMA,
        pltpu.SemaphoreType.DMA,
    ],
)
def _sc_scatter(presup, src, dst, w, zeros, out,
                src_v, dst_v, w_v, rows_v, acc, sem_g, sem_i, sem_s):
    cid = lax.axis_index("c")
    sid = lax.axis_index("s")
    wid = cid * NS + sid

    pltpu.async_copy(zeros, acc.at[pl.ds(sid * RPT, RPT)], sem_s)
    pltpu.make_async_copy(zeros, acc.at[pl.ds(sid * RPT, RPT)], sem_s).wait()
    plsc.subcore_barrier()

    def issue_idx(j, b):
        pltpu.async_copy(src.at[wid, j], src_v.at[b], sem_i)
        pltpu.async_copy(dst.at[wid, j], dst_v.at[b], sem_i)
        pltpu.async_copy(w.at[wid, j], w_v.at[b], sem_i)

    def wait_idx(b):
        pltpu.make_async_copy(src.at[0, 0], src_v.at[b], sem_i).wait()
        pltpu.make_async_copy(dst.at[0, 0], dst_v.at[b], sem_i).wait()
        pltpu.make_async_copy(w.at[0, 0], w_v.at[b], sem_i).wait()

    def issue_gather(b):
        pltpu.async_copy(presup.at[src_v.at[b]], rows_v.at[b], sem_g)

    def wait_gather(b):
        pltpu.make_async_copy(presup.at[pl.ds(0, K)], rows_v.at[b],
                              sem_g).wait()

    def scale_scatter(b):
        def scale_body(k, c2):
            wv = w_v[b, k, :]
            for c in range(D // 16):
                sl = pl.ds(c * 16, 16)
                rows_v[b, k, sl] = rows_v[b, k, sl] * wv
            return c2
        lax.fori_loop(0, K, scale_body, 0)
        pltpu.sync_copy(rows_v.at[b], acc.at[dst_v.at[b]], add=True)

    issue_idx(0, 0)
    wait_idx(0)
    issue_gather(0)
    issue_idx(1, 1)

    def pair_body(t, carry):
        j0 = 2 * t
        wait_idx(1)
        issue_gather(1)
        wait_gather(0)
        scale_scatter(0)
        issue_idx(j0 + 2, 0)
        wait_idx(0)
        issue_gather(0)
        wait_gather(1)
        scale_scatter(1)

        @pl.when(j0 + 3 < CH)
        def _():
            issue_idx(j0 + 3, 1)
        return carry
    lax.fori_loop(0, (CH - 1) // 2, pair_body, 0)

    wait_gather(0)
    scale_scatter(0)

    plsc.subcore_barrier()
    pltpu.sync_copy(acc.at[pl.ds(sid * RPT, RPT)],
                    out.at[cid, pl.ds(sid * RPT, RPT)])


def kernel(x, edge_index, edge_weight, W):
    pre_sup = pl.pallas_call(
        _mm_body,
        grid=(N // MB,),
        in_specs=[
            pl.BlockSpec((MB, D), lambda i: (i, 0)),
            pl.BlockSpec((D, D), lambda i: (0, 0)),
        ],
        out_specs=pl.BlockSpec((MB, D), lambda i: (i, 0)),
        out_shape=jax.ShapeDtypeStruct((N, D), jnp.float32),
    )(x, W)

    src = edge_index[0].astype(jnp.int32).reshape(NW, CH, K)
    dst = edge_index[1].astype(jnp.int32).reshape(NW, CH, K)
    zeros = jnp.zeros((RPT, D), jnp.float32)
    w_splat = jnp.broadcast_to(
        edge_weight.astype(jnp.float32).reshape(NW, CH, K)[..., None],
        (NW, CH, K, 16))

    partials = _sc_scatter(pre_sup, src, dst, w_splat, zeros)

    out = pl.pallas_call(
        _combine_body,
        grid=(N // MB,),
        in_specs=[
            pl.BlockSpec((1, MB, D), lambda i: (0, i, 0)),
            pl.BlockSpec((1, MB, D), lambda i: (1, i, 0)),
        ],
        out_specs=pl.BlockSpec((MB, D), lambda i: (i, 0)),
        out_shape=jax.ShapeDtypeStruct((N, D), jnp.float32),
    )(partials, partials)
    return out
